# Initial kernel scaffold; baseline (speedup 1.0000x reference)
import jax
import jax.numpy as jnp
from jax.experimental import pallas as pl


def kernel(word_embeddings, pos_table, ln_weight, ln_bias):
    raise NotImplementedError("write your pallas kernel here")


# speedup vs baseline: 2.1448x; 2.1448x over previous
import jax
import jax.numpy as jnp
from jax.experimental import pallas as pl

_EPS = 1e-08
_BLK = 512


def _body(w_ref, p_ref, g_ref, b_ref, o_ref):
    x = w_ref[0] + p_ref[...]
    mean = jnp.mean(x, axis=-1, keepdims=True)
    xc = x - mean
    var = jnp.mean(xc * xc, axis=-1, keepdims=True)
    normed = xc * jax.lax.rsqrt(var + _EPS)
    o_ref[0] = normed * g_ref[...] + b_ref[...]


def kernel(word_embeddings, pos_table, ln_weight, ln_bias):
    B, L, H = word_embeddings.shape
    pos = jax.lax.slice(pos_table, (0, 0), (L, H))
    grid = (L // _BLK, B)
    return pl.pallas_call(
        _body,
        grid=grid,
        in_specs=[
            pl.BlockSpec((1, _BLK, H), lambda i, b: (b, i, 0)),
            pl.BlockSpec((_BLK, H), lambda i, b: (i, 0)),
            pl.BlockSpec((1, H), lambda i, b: (0, 0)),
            pl.BlockSpec((1, H), lambda i, b: (0, 0)),
        ],
        out_specs=pl.BlockSpec((1, _BLK, H), lambda i, b: (b, i, 0)),
        out_shape=jax.ShapeDtypeStruct((B, L, H), jnp.float32),
    )(word_embeddings, pos, ln_weight.reshape(1, H), ln_bias.reshape(1, H))


# speedup vs baseline: 2.4558x; 1.1450x over previous
import jax
import jax.numpy as jnp
from jax.experimental import pallas as pl

_EPS = 1e-08
_BLK = 1024


def _body(w_ref, p_ref, g_ref, b_ref, o_ref):
    x = w_ref[0] + p_ref[...]
    mean = jnp.mean(x, axis=-1, keepdims=True)
    xc = x - mean
    var = jnp.mean(xc * xc, axis=-1, keepdims=True)
    normed = xc * jax.lax.rsqrt(var + _EPS)
    o_ref[0] = normed * g_ref[...] + b_ref[...]


def kernel(word_embeddings, pos_table, ln_weight, ln_bias):
    B, L, H = word_embeddings.shape
    pos = jax.lax.slice(pos_table, (0, 0), (L, H))
    grid = (L // _BLK, B)
    return pl.pallas_call(
        _body,
        grid=grid,
        in_specs=[
            pl.BlockSpec((1, _BLK, H), lambda i, b: (b, i, 0)),
            pl.BlockSpec((_BLK, H), lambda i, b: (i, 0)),
            pl.BlockSpec((1, H), lambda i, b: (0, 0)),
            pl.BlockSpec((1, H), lambda i, b: (0, 0)),
        ],
        out_specs=pl.BlockSpec((1, _BLK, H), lambda i, b: (b, i, 0)),
        out_shape=jax.ShapeDtypeStruct((B, L, H), jnp.float32),
    )(word_embeddings, pos, ln_weight.reshape(1, H), ln_bias.reshape(1, H))


# speedup vs baseline: 2.5814x; 1.0511x over previous
import jax
import jax.numpy as jnp
from jax.experimental import pallas as pl

_EPS = 1e-08
_BLK = 2048


def _body(w_ref, p_ref, g_ref, b_ref, o_ref):
    x = w_ref[0] + p_ref[...]
    mean = jnp.mean(x, axis=-1, keepdims=True)
    xc = x - mean
    var = jnp.mean(xc * xc, axis=-1, keepdims=True)
    normed = xc * jax.lax.rsqrt(var + _EPS)
    o_ref[0] = normed * g_ref[...] + b_ref[...]


def kernel(word_embeddings, pos_table, ln_weight, ln_bias):
    B, L, H = word_embeddings.shape
    pos = jax.lax.slice(pos_table, (0, 0), (L, H))
    grid = (L // _BLK, B)
    return pl.pallas_call(
        _body,
        grid=grid,
        in_specs=[
            pl.BlockSpec((1, _BLK, H), lambda i, b: (b, i, 0)),
            pl.BlockSpec((_BLK, H), lambda i, b: (i, 0)),
            pl.BlockSpec((1, H), lambda i, b: (0, 0)),
            pl.BlockSpec((1, H), lambda i, b: (0, 0)),
        ],
        out_specs=pl.BlockSpec((1, _BLK, H), lambda i, b: (b, i, 0)),
        out_shape=jax.ShapeDtypeStruct((B, L, H), jnp.float32),
    )(word_embeddings, pos, ln_weight.reshape(1, H), ln_bias.reshape(1, H))


# speedup vs baseline: 2.7275x; 1.0566x over previous
import jax
import jax.numpy as jnp
from jax.experimental import pallas as pl

_EPS = 1e-08
_BLK = 512


def _body(w_ref, p_ref, g_ref, b_ref, o_ref):
    x = w_ref[...] + p_ref[...][None]
    mean = jnp.mean(x, axis=-1, keepdims=True)
    xc = x - mean
    var = jnp.mean(xc * xc, axis=-1, keepdims=True)
    normed = xc * jax.lax.rsqrt(var + _EPS)
    o_ref[...] = normed * g_ref[...] + b_ref[...]


def kernel(word_embeddings, pos_table, ln_weight, ln_bias):
    B, L, H = word_embeddings.shape
    pos = jax.lax.slice(pos_table, (0, 0), (L, H))
    grid = (L // _BLK,)
    return pl.pallas_call(
        _body,
        grid=grid,
        in_specs=[
            pl.BlockSpec((B, _BLK, H), lambda i: (0, i, 0)),
            pl.BlockSpec((_BLK, H), lambda i: (i, 0)),
            pl.BlockSpec((1, H), lambda i: (0, 0)),
            pl.BlockSpec((1, H), lambda i: (0, 0)),
        ],
        out_specs=pl.BlockSpec((B, _BLK, H), lambda i: (0, i, 0)),
        out_shape=jax.ShapeDtypeStruct((B, L, H), jnp.float32),
    )(word_embeddings, pos, ln_weight.reshape(1, H), ln_bias.reshape(1, H))
